# baseline (speedup 1.0000x reference)
import functools

import jax
import jax.numpy as jnp
from jax import lax
from jax.experimental import pallas as pl
from jax.experimental.pallas import tpu as pltpu
from jax.experimental.pallas import tpu_sc as plsc

N_NODES = 50000
D = 256
B = 4096
L = 20
EV = 64
BL = B * L

MM_BLK = 1000
N_MM = N_NODES // MM_BLK
TAB_ROWS = 51000
ZERO_ROW = 50072

G_BLK = 512
N_G = BL // G_BLK
S_SC = 20480
S_OFF = S_SC // G_BLK

NW = 32
PW = S_SC // NW
C = 64
NCH = PW // C
NBUF = 2
NPH = NCH // NBUF


def _table_body(a_ref, e_ref, w_ref, out_ref):
    s = pl.program_id(0)

    @pl.when(s < N_MM)
    def _():
        out_ref[...] = jnp.maximum(
            jnp.dot(a_ref[...], w_ref[...],
                    preferred_element_type=jnp.float32), 0.0)

    @pl.when(s == N_MM)
    def _():
        out_ref[0:EV, :] = jnp.maximum(
            jnp.dot(e_ref[...], w_ref[...], preferred_element_type=jnp.float32,
                    precision=lax.Precision.HIGHEST), 0.0)
        out_ref[EV:, :] = jnp.zeros((MM_BLK - EV, D), jnp.float32)


def _build_table(a, e, w):
    return pl.pallas_call(
        _table_body,
        grid=(N_MM + 1,),
        in_specs=[
            pl.BlockSpec((MM_BLK, D), lambda s: (jnp.minimum(s, N_MM - 1), 0)),
            pl.BlockSpec((EV, D), lambda s: (0, 0)),
            pl.BlockSpec((D, D), lambda s: (0, 0)),
        ],
        out_specs=pl.BlockSpec((MM_BLK, D), lambda s: (s, 0)),
        out_shape=jax.ShapeDtypeStruct((TAB_ROWS, D), jnp.float32),
    )(a, e, w)


_mesh = plsc.VectorSubcoreMesh(core_axis_name="c", subcore_axis_name="s")


@functools.partial(
    pl.kernel,
    mesh=_mesh,
    out_type=jax.ShapeDtypeStruct((S_SC, D), jnp.float32),
    scratch_types=[
        pltpu.VMEM((PW,), jnp.int32),
        pltpu.VMEM((PW,), jnp.int32),
        pltpu.VMEM((NCH, C), jnp.int32),
        pltpu.VMEM((NBUF, C, D), jnp.float32),
        pltpu.SemaphoreType.DMA,
        pltpu.SemaphoreType.DMA,
        pltpu.SemaphoreType.DMA,
        pltpu.SemaphoreType.DMA,
    ],
)
def _sc_gather(tb, idxh, thrh, n_out, idx_v, thr_v, gn_v, bufs,
               g0, g1, s0, s1):
    wid = lax.axis_index("s") * 2 + lax.axis_index("c")
    base = wid * PW

    pltpu.sync_copy(idxh.at[pl.ds(base, PW)], idx_v)
    pltpu.sync_copy(thrh.at[pl.ds(base, PW)], thr_v)

    lane = lax.iota(jnp.int32, 16)

    def mask_body(k, carry):
        for j in range(C // 16):
            p0 = k * C + j * 16
            gflat = base + p0 + lane
            m = gflat < thr_v[pl.ds(p0, 16)]
            gn_v[k, pl.ds(j * 16, 16)] = jnp.where(
                m, idx_v[pl.ds(p0, 16)], ZERO_ROW)
        return carry

    lax.fori_loop(0, NCH, mask_body, 0)

    gsems = (g0, g1)
    ssems = (s0, s1)
    for b in range(NBUF):
        pltpu.async_copy(tb.at[gn_v.at[b]], bufs.at[b], gsems[b])

    def phase(p, carry):
        k0 = p * NBUF
        for b in range(NBUF):
            off = (k0 + b) * C
            pltpu.make_async_copy(tb.at[gn_v.at[0]], bufs.at[b],
                                  gsems[b]).wait()
            pltpu.async_copy(bufs.at[b], n_out.at[pl.ds(base + off, C)],
                             ssems[b])
        for b in range(NBUF):
            nk = k0 + b + NBUF
            pltpu.make_async_copy(bufs.at[b], n_out.at[pl.ds(base, C)],
                                  ssems[b]).wait()

            @pl.when(nk < NCH)
            def _():
                pltpu.async_copy(tb.at[gn_v.at[nk]], bufs.at[b], gsems[b])
        return carry

    lax.fori_loop(0, NPH, phase, 0)


def _fused_body(a_ref, e_ref, w_ref, idxn_ref, et_ref, thr_ref,
                out_n_ref, out_e_ref, tab_ref):
    s = pl.program_id(0)

    @pl.when(s < N_MM)
    def _():
        tab_ref[pl.ds(s * MM_BLK, MM_BLK), :] = jnp.maximum(
            jnp.dot(a_ref[...], w_ref[...],
                    preferred_element_type=jnp.float32), 0.0)

    @pl.when(s == N_MM)
    def _():
        tab_ref[pl.ds(N_NODES, EV), :] = jnp.maximum(
            jnp.dot(e_ref[...], w_ref[...], preferred_element_type=jnp.float32,
                    precision=lax.Precision.HIGHEST), 0.0)

    @pl.when(s > N_MM)
    def _():
        g = s - (N_MM + 1)

        pos = (g * G_BLK
               + lax.broadcasted_iota(jnp.int32, (G_BLK, 1), 0))
        live = pos < thr_ref[0]

        @pl.when(g >= S_OFF)
        def _():
            def grp(j, carry):
                rows = [tab_ref[pl.ds(idxn_ref[0, 0, j * 8 + k], 1), :]
                        for k in range(8)]
                out_n_ref[pl.ds(j * 8, 8), :] = jnp.concatenate(rows, axis=0)
                return carry

            lax.fori_loop(0, G_BLK // 8, grp, 0, unroll=2)
            out_n_ref[...] = jnp.where(live, out_n_ref[...], 0.0)

        onehot = jnp.where(
            et_ref[0] == lax.broadcasted_iota(jnp.int32, (G_BLK, EV), 1),
            1.0, 0.0)
        te = tab_ref[pl.ds(N_NODES, EV), :]
        eo = jnp.dot(onehot, te, preferred_element_type=jnp.float32,
                     precision=lax.Precision.HIGHEST)
        out_e_ref[...] = jnp.where(live, eo, 0.0)


def kernel(all_nodes_encodings, paths_nodes_indices, paths_edge_types,
           paths_lengths, edge_types_embeddings, W_seq):
    idx_flat = paths_nodes_indices.reshape(BL).astype(jnp.int32)
    idx = idx_flat.reshape(N_G, 1, G_BLK)
    et = paths_edge_types.reshape(N_G, G_BLK, 1).astype(jnp.int32)
    thr_flat = jnp.repeat(jnp.arange(B, dtype=jnp.int32) * L
                          + paths_lengths.astype(jnp.int32), L)
    thr = thr_flat.reshape(N_G, G_BLK, 1)

    table = _build_table(all_nodes_encodings, edge_types_embeddings, W_seq)
    sc_nodes = _sc_gather(table, idx_flat, thr_flat)

    grid = N_MM + 1 + N_G
    tc_nodes, edges_flat = pl.pallas_call(
        _fused_body,
        grid=(grid,),
        in_specs=[
            pl.BlockSpec((MM_BLK, D), lambda s: (jnp.minimum(s, N_MM - 1), 0)),
            pl.BlockSpec((EV, D), lambda s: (0, 0)),
            pl.BlockSpec((D, D), lambda s: (0, 0)),
            pl.BlockSpec((1, 1, G_BLK),
                         lambda s: (jnp.maximum(s - N_MM - 1, 0), 0, 0),
                         memory_space=pltpu.SMEM),
            pl.BlockSpec((1, G_BLK, 1), lambda s: (jnp.maximum(s - N_MM - 1, 0),
                                                   0, 0)),
            pl.BlockSpec((1, G_BLK, 1), lambda s: (jnp.maximum(s - N_MM - 1, 0),
                                                   0, 0)),
        ],
        out_specs=[
            pl.BlockSpec((G_BLK, D),
                         lambda s: (jnp.maximum(s - N_MM - 1 - S_OFF, 0), 0)),
            pl.BlockSpec((G_BLK, D), lambda s: (jnp.maximum(s - N_MM - 1, 0), 0)),
        ],
        out_shape=[
            jax.ShapeDtypeStruct((BL - S_SC, D), jnp.float32),
            jax.ShapeDtypeStruct((BL, D), jnp.float32),
        ],
        scratch_shapes=[pltpu.VMEM((TAB_ROWS, D), jnp.float32)],
    )(all_nodes_encodings, edge_types_embeddings, W_seq, idx, et, thr)

    nodes_flat = jnp.concatenate([sc_nodes, tc_nodes], axis=0)
    return nodes_flat.reshape(B, L, D), edges_flat.reshape(B, L, D)


# speedup vs baseline: 1.5971x; 1.5971x over previous
import jax
import jax.numpy as jnp
from jax import lax
from jax.experimental import pallas as pl
from jax.experimental.pallas import tpu as pltpu

N_NODES = 50000
D = 256
B = 4096
L = 20
EV = 64
BL = B * L

MM_BLK = 1000
N_MM = N_NODES // MM_BLK
TAB_ROWS = 51000

G_BLK = 512
N_G = BL // G_BLK


def _fused_body(a_ref, e_ref, w_ref, idxn_ref, et_ref, thr_ref,
                out_n_ref, out_e_ref, tab_ref):
    s = pl.program_id(0)

    @pl.when(s < N_MM)
    def _():
        tab_ref[pl.ds(s * MM_BLK, MM_BLK), :] = jnp.maximum(
            jnp.dot(a_ref[...], w_ref[...],
                    preferred_element_type=jnp.float32), 0.0)

    @pl.when(s == N_MM)
    def _():
        tab_ref[pl.ds(N_NODES, EV), :] = jnp.maximum(
            jnp.dot(e_ref[...], w_ref[...], preferred_element_type=jnp.float32,
                    precision=lax.Precision.HIGHEST), 0.0)

    @pl.when(s > N_MM)
    def _():
        g = s - (N_MM + 1)

        def grp(j, carry):
            rows = [tab_ref[pl.ds(idxn_ref[0, 0, j * 8 + k], 1), :]
                    for k in range(8)]
            out_n_ref[pl.ds(j * 8, 8), :] = jnp.concatenate(rows, axis=0)
            return carry

        lax.fori_loop(0, G_BLK // 8, grp, 0, unroll=4)

        pos = (g * G_BLK
               + lax.broadcasted_iota(jnp.int32, (G_BLK, 1), 0))
        live = pos < thr_ref[0]
        out_n_ref[...] = jnp.where(live, out_n_ref[...], 0.0)

        onehot = jnp.where(
            et_ref[0] == lax.broadcasted_iota(jnp.int32, (G_BLK, EV), 1),
            1.0, 0.0)
        te = tab_ref[pl.ds(N_NODES, EV), :]
        eo = jnp.dot(onehot, te, preferred_element_type=jnp.float32)
        out_e_ref[...] = jnp.where(live, eo, 0.0)


def kernel(all_nodes_encodings, paths_nodes_indices, paths_edge_types,
           paths_lengths, edge_types_embeddings, W_seq):
    idx = paths_nodes_indices.reshape(N_G, 1, G_BLK).astype(jnp.int32)
    et = paths_edge_types.reshape(N_G, G_BLK, 1).astype(jnp.int32)
    thr = jnp.repeat(jnp.arange(B, dtype=jnp.int32) * L
                     + paths_lengths.astype(jnp.int32), L)
    thr = thr.reshape(N_G, G_BLK, 1)

    grid = N_MM + 1 + N_G
    nodes_flat, edges_flat = pl.pallas_call(
        _fused_body,
        grid=(grid,),
        in_specs=[
            pl.BlockSpec((MM_BLK, D), lambda s: (jnp.minimum(s, N_MM - 1), 0)),
            pl.BlockSpec((EV, D), lambda s: (0, 0)),
            pl.BlockSpec((D, D), lambda s: (0, 0)),
            pl.BlockSpec((1, 1, G_BLK),
                         lambda s: (jnp.maximum(s - N_MM - 1, 0), 0, 0),
                         memory_space=pltpu.SMEM),
            pl.BlockSpec((1, G_BLK, 1), lambda s: (jnp.maximum(s - N_MM - 1, 0),
                                                   0, 0)),
            pl.BlockSpec((1, G_BLK, 1), lambda s: (jnp.maximum(s - N_MM - 1, 0),
                                                   0, 0)),
        ],
        out_specs=[
            pl.BlockSpec((G_BLK, D), lambda s: (jnp.maximum(s - N_MM - 1, 0), 0)),
            pl.BlockSpec((G_BLK, D), lambda s: (jnp.maximum(s - N_MM - 1, 0), 0)),
        ],
        out_shape=[
            jax.ShapeDtypeStruct((BL, D), jnp.float32),
            jax.ShapeDtypeStruct((BL, D), jnp.float32),
        ],
        scratch_shapes=[pltpu.VMEM((TAB_ROWS, D), jnp.float32)],
    )(all_nodes_encodings, edge_types_embeddings, W_seq, idx, et, thr)
    return nodes_flat.reshape(B, L, D), edges_flat.reshape(B, L, D)


# speedup vs baseline: 1.6436x; 1.0291x over previous
import jax
import jax.numpy as jnp
from jax import lax
from jax.experimental import pallas as pl
from jax.experimental.pallas import tpu as pltpu

N_NODES = 50000
D = 256
B = 4096
L = 20
EV = 64
BL = B * L

MM_BLK = 1000
N_MM = N_NODES // MM_BLK
TAB_ROWS = 50176

G_BLK = 1024
N_G = BL // G_BLK


def _fused_body(a_ref, e_ref, w_ref, idxn_ref, et_ref, thr_ref,
                out_n_ref, out_e_ref, tab_ref):
    s = pl.program_id(0)

    @pl.when(s < N_MM)
    def _():
        tab_ref[pl.ds(s * MM_BLK, MM_BLK), :] = jnp.maximum(
            jnp.dot(a_ref[...], w_ref[...],
                    preferred_element_type=jnp.float32), 0.0)

    @pl.when(s == N_MM)
    def _():
        tab_ref[pl.ds(N_NODES, EV), :] = jnp.maximum(
            jnp.dot(e_ref[...], w_ref[...], preferred_element_type=jnp.float32,
                    precision=lax.Precision.HIGHEST), 0.0)

    @pl.when(s > N_MM)
    def _():
        g = s - (N_MM + 1)

        def grp(j, carry):
            rows = [tab_ref[pl.ds(idxn_ref[0, 0, j * 8 + k], 1), :]
                    for k in range(8)]
            out_n_ref[pl.ds(j * 8, 8), :] = jnp.concatenate(rows, axis=0)
            return carry

        lax.fori_loop(0, G_BLK // 8, grp, 0, unroll=4)

        pos = (g * G_BLK
               + lax.broadcasted_iota(jnp.int32, (G_BLK, 1), 0))
        live = pos < thr_ref[0]
        out_n_ref[...] = jnp.where(live, out_n_ref[...], 0.0)

        onehot = jnp.where(
            et_ref[0] == lax.broadcasted_iota(jnp.int32, (G_BLK, EV), 1),
            1.0, 0.0)
        te = tab_ref[pl.ds(N_NODES, EV), :]
        eo = jnp.dot(onehot, te, preferred_element_type=jnp.float32)
        out_e_ref[...] = jnp.where(live, eo, 0.0)


def kernel(all_nodes_encodings, paths_nodes_indices, paths_edge_types,
           paths_lengths, edge_types_embeddings, W_seq):
    idx = paths_nodes_indices.reshape(N_G, 1, G_BLK).astype(jnp.int32)
    et = paths_edge_types.reshape(N_G, G_BLK, 1).astype(jnp.int32)
    thr = jnp.repeat(jnp.arange(B, dtype=jnp.int32) * L
                     + paths_lengths.astype(jnp.int32), L)
    thr = thr.reshape(N_G, G_BLK, 1)

    grid = N_MM + 1 + N_G
    nodes_flat, edges_flat = pl.pallas_call(
        _fused_body,
        grid=(grid,),
        in_specs=[
            pl.BlockSpec((MM_BLK, D), lambda s: (jnp.minimum(s, N_MM - 1), 0)),
            pl.BlockSpec((EV, D), lambda s: (0, 0)),
            pl.BlockSpec((D, D), lambda s: (0, 0)),
            pl.BlockSpec((1, 1, G_BLK),
                         lambda s: (jnp.maximum(s - N_MM - 1, 0), 0, 0),
                         memory_space=pltpu.SMEM),
            pl.BlockSpec((1, G_BLK, 1), lambda s: (jnp.maximum(s - N_MM - 1, 0),
                                                   0, 0)),
            pl.BlockSpec((1, G_BLK, 1), lambda s: (jnp.maximum(s - N_MM - 1, 0),
                                                   0, 0)),
        ],
        out_specs=[
            pl.BlockSpec((G_BLK, D), lambda s: (jnp.maximum(s - N_MM - 1, 0), 0)),
            pl.BlockSpec((G_BLK, D), lambda s: (jnp.maximum(s - N_MM - 1, 0), 0)),
        ],
        out_shape=[
            jax.ShapeDtypeStruct((BL, D), jnp.float32),
            jax.ShapeDtypeStruct((BL, D), jnp.float32),
        ],
        scratch_shapes=[pltpu.VMEM((TAB_ROWS, D), jnp.float32)],
    )(all_nodes_encodings, edge_types_embeddings, W_seq, idx, et, thr)
    return nodes_flat.reshape(B, L, D), edges_flat.reshape(B, L, D)


# speedup vs baseline: 1.6581x; 1.0088x over previous
import jax
import jax.numpy as jnp
from jax import lax
from jax.experimental import pallas as pl
from jax.experimental.pallas import tpu as pltpu

N_NODES = 50000
D = 256
B = 4096
L = 20
EV = 64
BL = B * L

MM_BLK = 1000
N_MM = N_NODES // MM_BLK
TAB_ROWS = 50176

G_BLK = 1024
N_G = BL // G_BLK


def _fused_body(a_ref, e_ref, w_ref, idxn_ref, et_ref, thr_ref,
                out_n_ref, out_e_ref, tab_ref):
    s = pl.program_id(0)

    @pl.when(s < N_MM)
    def _():
        tab_ref[pl.ds(s * MM_BLK, MM_BLK), :] = jnp.maximum(
            jnp.dot(a_ref[...], w_ref[...],
                    preferred_element_type=jnp.float32), 0.0)

    @pl.when(s == N_MM)
    def _():
        tab_ref[pl.ds(N_NODES, EV), :] = jnp.maximum(
            jnp.dot(e_ref[...], w_ref[...], preferred_element_type=jnp.float32,
                    precision=lax.Precision.HIGHEST), 0.0)

    @pl.when(s > N_MM)
    def _():
        g = s - (N_MM + 1)

        def grp(j, carry):
            rows = [tab_ref[pl.ds(idxn_ref[0, 0, j * 8 + k], 1), :]
                    for k in range(8)]
            out_n_ref[pl.ds(j * 8, 8), :] = jnp.concatenate(rows, axis=0)
            return carry

        lax.fori_loop(0, G_BLK // 8, grp, 0, unroll=8)

        pos = (g * G_BLK
               + lax.broadcasted_iota(jnp.int32, (G_BLK, 1), 0))
        live = pos < thr_ref[0]
        out_n_ref[...] = jnp.where(live, out_n_ref[...], 0.0)

        onehot = jnp.where(
            et_ref[0] == lax.broadcasted_iota(jnp.int32, (G_BLK, EV), 1),
            1.0, 0.0)
        te = tab_ref[pl.ds(N_NODES, EV), :]
        eo = jnp.dot(onehot, te, preferred_element_type=jnp.float32)
        out_e_ref[...] = jnp.where(live, eo, 0.0)


def kernel(all_nodes_encodings, paths_nodes_indices, paths_edge_types,
           paths_lengths, edge_types_embeddings, W_seq):
    idx = paths_nodes_indices.reshape(N_G, 1, G_BLK).astype(jnp.int32)
    et = paths_edge_types.reshape(N_G, G_BLK, 1).astype(jnp.int32)
    thr = jnp.repeat(jnp.arange(B, dtype=jnp.int32) * L
                     + paths_lengths.astype(jnp.int32), L)
    thr = thr.reshape(N_G, G_BLK, 1)

    grid = N_MM + 1 + N_G
    nodes_flat, edges_flat = pl.pallas_call(
        _fused_body,
        grid=(grid,),
        in_specs=[
            pl.BlockSpec((MM_BLK, D), lambda s: (jnp.minimum(s, N_MM - 1), 0)),
            pl.BlockSpec((EV, D), lambda s: (0, 0)),
            pl.BlockSpec((D, D), lambda s: (0, 0)),
            pl.BlockSpec((1, 1, G_BLK),
                         lambda s: (jnp.maximum(s - N_MM - 1, 0), 0, 0),
                         memory_space=pltpu.SMEM),
            pl.BlockSpec((1, G_BLK, 1), lambda s: (jnp.maximum(s - N_MM - 1, 0),
                                                   0, 0)),
            pl.BlockSpec((1, G_BLK, 1), lambda s: (jnp.maximum(s - N_MM - 1, 0),
                                                   0, 0)),
        ],
        out_specs=[
            pl.BlockSpec((G_BLK, D), lambda s: (jnp.maximum(s - N_MM - 1, 0), 0)),
            pl.BlockSpec((G_BLK, D), lambda s: (jnp.maximum(s - N_MM - 1, 0), 0)),
        ],
        out_shape=[
            jax.ShapeDtypeStruct((BL, D), jnp.float32),
            jax.ShapeDtypeStruct((BL, D), jnp.float32),
        ],
        scratch_shapes=[pltpu.VMEM((TAB_ROWS, D), jnp.float32)],
    )(all_nodes_encodings, edge_types_embeddings, W_seq, idx, et, thr)
    return nodes_flat.reshape(B, L, D), edges_flat.reshape(B, L, D)


# speedup vs baseline: 1.6617x; 1.0022x over previous
import jax
import jax.numpy as jnp
from jax import lax
from jax.experimental import pallas as pl
from jax.experimental.pallas import tpu as pltpu

N_NODES = 50000
D = 256
B = 4096
L = 20
EV = 64
BL = B * L

MM_BLK = 1000
N_MM = N_NODES // MM_BLK
TAB_ROWS = 50176

G_BLK = 1024
N_G = BL // G_BLK


def _fused_body(a_ref, e_ref, w_ref, idxn_ref, et_ref, thr_ref,
                out_n_ref, out_e_ref, tab_ref):
    s = pl.program_id(0)

    @pl.when(s < N_MM)
    def _():
        tab_ref[pl.ds(s * MM_BLK, MM_BLK), :] = jnp.maximum(
            jnp.dot(a_ref[...], w_ref[...],
                    preferred_element_type=jnp.float32), 0.0)

    @pl.when(s == N_MM)
    def _():
        tab_ref[pl.ds(N_NODES, EV), :] = jnp.maximum(
            jnp.dot(e_ref[...], w_ref[...], preferred_element_type=jnp.float32,
                    precision=lax.Precision.HIGHEST), 0.0)

    @pl.when(s > N_MM)
    def _():
        g = s - (N_MM + 1)

        def grp(j, carry):
            rows = [tab_ref[pl.ds(idxn_ref[0, 0, j * 16 + k], 1), :]
                    for k in range(16)]
            out_n_ref[pl.ds(j * 16, 16), :] = jnp.concatenate(rows, axis=0)
            return carry

        lax.fori_loop(0, G_BLK // 16, grp, 0, unroll=4)

        pos = (g * G_BLK
               + lax.broadcasted_iota(jnp.int32, (G_BLK, 1), 0))
        live = pos < thr_ref[0]
        out_n_ref[...] = jnp.where(live, out_n_ref[...], 0.0)

        onehot = jnp.where(
            et_ref[0] == lax.broadcasted_iota(jnp.int32, (G_BLK, EV), 1),
            1.0, 0.0)
        te = tab_ref[pl.ds(N_NODES, EV), :]
        eo = jnp.dot(onehot, te, preferred_element_type=jnp.float32)
        out_e_ref[...] = jnp.where(live, eo, 0.0)


def kernel(all_nodes_encodings, paths_nodes_indices, paths_edge_types,
           paths_lengths, edge_types_embeddings, W_seq):
    idx = paths_nodes_indices.reshape(N_G, 1, G_BLK).astype(jnp.int32)
    et = paths_edge_types.reshape(N_G, G_BLK, 1).astype(jnp.int32)
    thr = jnp.repeat(jnp.arange(B, dtype=jnp.int32) * L
                     + paths_lengths.astype(jnp.int32), L)
    thr = thr.reshape(N_G, G_BLK, 1)

    grid = N_MM + 1 + N_G
    nodes_flat, edges_flat = pl.pallas_call(
        _fused_body,
        grid=(grid,),
        in_specs=[
            pl.BlockSpec((MM_BLK, D), lambda s: (jnp.minimum(s, N_MM - 1), 0)),
            pl.BlockSpec((EV, D), lambda s: (0, 0)),
            pl.BlockSpec((D, D), lambda s: (0, 0)),
            pl.BlockSpec((1, 1, G_BLK),
                         lambda s: (jnp.maximum(s - N_MM - 1, 0), 0, 0),
                         memory_space=pltpu.SMEM),
            pl.BlockSpec((1, G_BLK, 1), lambda s: (jnp.maximum(s - N_MM - 1, 0),
                                                   0, 0)),
            pl.BlockSpec((1, G_BLK, 1), lambda s: (jnp.maximum(s - N_MM - 1, 0),
                                                   0, 0)),
        ],
        out_specs=[
            pl.BlockSpec((G_BLK, D), lambda s: (jnp.maximum(s - N_MM - 1, 0), 0)),
            pl.BlockSpec((G_BLK, D), lambda s: (jnp.maximum(s - N_MM - 1, 0), 0)),
        ],
        out_shape=[
            jax.ShapeDtypeStruct((BL, D), jnp.float32),
            jax.ShapeDtypeStruct((BL, D), jnp.float32),
        ],
        scratch_shapes=[pltpu.VMEM((TAB_ROWS, D), jnp.float32)],
    )(all_nodes_encodings, edge_types_embeddings, W_seq, idx, et, thr)
    return nodes_flat.reshape(B, L, D), edges_flat.reshape(B, L, D)
